# baseline (speedup 1.0000x reference)
import jax
import jax.numpy as jnp
from jax import lax
from jax.experimental import pallas as pl
from jax.experimental.pallas import tpu as pltpu
from jax.experimental.pallas import tpu_sc as plsc

NUM_EMB = 1000000
DIM = 128
B = 4096
L = 50
N = B * L

NC, NS = 2, 16
NW = NC * NS
K = 2
NK = N // K
ROWS_PER_W = NK // NW
CHUNK = 128
NCHUNK = ROWS_PER_W // CHUNK
NBUF = 6


def _sc_gather(ids_hbm, table_hbm, out_hbm, idx_v, rows_v, g_sem, s_sem):
    wid = lax.axis_index("s") * NC + lax.axis_index("c")
    base = wid * ROWS_PER_W
    pltpu.sync_copy(ids_hbm.at[wid], idx_v)
    g_h = [None] * NCHUNK
    s_h = [None] * NCHUNK
    for c in range(NCHUNK + NBUF - 1):
        if c < NCHUNK:
            if c >= NBUF:
                s_h[c - NBUF].wait()
            g_h[c] = pltpu.async_copy(
                table_hbm.at[idx_v.at[c]], rows_v.at[c % NBUF], g_sem
            )
        d = c - (NBUF - 1)
        if 0 <= d < NCHUNK:
            g_h[d].wait()
            s_h[d] = pltpu.async_copy(
                rows_v.at[d % NBUF],
                out_hbm.at[pl.ds(base + d * CHUNK, CHUNK)],
                s_sem,
            )
    for d in range(max(NCHUNK - NBUF, 0), NCHUNK):
        s_h[d].wait()


def _gather_call(ids, table):
    return pl.kernel(
        _sc_gather,
        mesh=plsc.VectorSubcoreMesh(
            core_axis_name="c", subcore_axis_name="s", num_cores=NC
        ),
        out_type=jax.ShapeDtypeStruct((NK, DIM), jnp.float32),
        scratch_types=[
            pltpu.VMEM((NCHUNK, CHUNK), jnp.int32),
            pltpu.VMEM((NBUF, CHUNK, DIM), jnp.float32),
            pltpu.SemaphoreType.DMA,
            pltpu.SemaphoreType.DMA,
        ],
    )(ids, table)


MM_BLK = 2048
MM_STEPS = NK // MM_BLK


def _mm_body(prev_ref, emb_ref, wt_ref, b_ref, out_ref):
    del prev_ref
    out_ref[...] = (
        jnp.dot(emb_ref[...], wt_ref[...], preferred_element_type=jnp.float32)
        + b_ref[...]
    )


def _tc_project_chunk(k, out_prev, emb_k, Wt, b2d):
    return pl.pallas_call(
        _mm_body,
        grid=(MM_STEPS,),
        in_specs=[
            pl.BlockSpec(memory_space=pl.ANY),
            pl.BlockSpec((MM_BLK, DIM), lambda i: (i, 0)),
            pl.BlockSpec((DIM, DIM), lambda i: (0, 0)),
            pl.BlockSpec((1, DIM), lambda i: (0, 0)),
        ],
        out_specs=pl.BlockSpec((MM_BLK, DIM), lambda i, k=k: (k * MM_STEPS + i, 0)),
        out_shape=jax.ShapeDtypeStruct((N, DIM), jnp.float32),
        input_output_aliases={0: 0},
    )(out_prev, emb_k, Wt, b2d)


def _mm_first_body(emb_ref, wt_ref, b_ref, out_ref):
    out_ref[...] = (
        jnp.dot(emb_ref[...], wt_ref[...], preferred_element_type=jnp.float32)
        + b_ref[...]
    )


def _tc_project_first(emb_k, Wt, b2d):
    return pl.pallas_call(
        _mm_first_body,
        grid=(MM_STEPS,),
        in_specs=[
            pl.BlockSpec((MM_BLK, DIM), lambda i: (i, 0)),
            pl.BlockSpec((DIM, DIM), lambda i: (0, 0)),
            pl.BlockSpec((1, DIM), lambda i: (0, 0)),
        ],
        out_specs=pl.BlockSpec((MM_BLK, DIM), lambda i: (i, 0)),
        out_shape=jax.ShapeDtypeStruct((N, DIM), jnp.float32),
    )(emb_k, Wt, b2d)


def kernel(news_ids, news_categ, table, W, b):
    del news_categ
    ids = news_ids.T.reshape(K, NW, NCHUNK, CHUNK).astype(jnp.int32)
    Wt = W.T
    b2d = b.reshape(1, DIM)
    embs = [_gather_call(ids[k], table) for k in range(K)]
    out = _tc_project_first(embs[0], Wt, b2d)
    for k in range(1, K):
        out = _tc_project_chunk(k, out, embs[k], Wt, b2d)
    return out.reshape(L, B, DIM).transpose(1, 0, 2)


# speedup vs baseline: 1.1181x; 1.1181x over previous
import jax
import jax.numpy as jnp
from jax import lax
from jax.experimental import pallas as pl
from jax.experimental.pallas import tpu as pltpu
from jax.experimental.pallas import tpu_sc as plsc

NUM_EMB = 1000000
DIM = 128
B = 4096
L = 50
N = B * L

NC, NS = 2, 16
NW = NC * NS
K = 5
NK = N // K
ROWS_PER_W = NK // NW
CHUNK = 128
NCHUNK = ROWS_PER_W // CHUNK
NBUF = 6


def _sc_gather(ids_hbm, table_hbm, out_hbm, idx_v, rows_v, g_sem, s_sem):
    wid = lax.axis_index("s") * NC + lax.axis_index("c")
    base = wid * ROWS_PER_W
    pltpu.sync_copy(ids_hbm.at[wid], idx_v)
    g_h = [None] * NCHUNK
    s_h = [None] * NCHUNK
    for c in range(NCHUNK + NBUF - 1):
        if c < NCHUNK:
            if c >= NBUF:
                s_h[c - NBUF].wait()
            g_h[c] = pltpu.async_copy(
                table_hbm.at[idx_v.at[c]], rows_v.at[c % NBUF], g_sem
            )
        d = c - (NBUF - 1)
        if 0 <= d < NCHUNK:
            g_h[d].wait()
            s_h[d] = pltpu.async_copy(
                rows_v.at[d % NBUF],
                out_hbm.at[pl.ds(base + d * CHUNK, CHUNK)],
                s_sem,
            )
    for d in range(max(NCHUNK - NBUF, 0), NCHUNK):
        s_h[d].wait()


def _gather_call(ids, table):
    return pl.kernel(
        _sc_gather,
        mesh=plsc.VectorSubcoreMesh(
            core_axis_name="c", subcore_axis_name="s", num_cores=NC
        ),
        out_type=jax.ShapeDtypeStruct((NK, DIM), jnp.float32),
        scratch_types=[
            pltpu.VMEM((NCHUNK, CHUNK), jnp.int32),
            pltpu.VMEM((NBUF, CHUNK, DIM), jnp.float32),
            pltpu.SemaphoreType.DMA,
            pltpu.SemaphoreType.DMA,
        ],
    )(ids, table)


MM_BLK = 4096
MM_STEPS = NK // MM_BLK


def _mm_body(prev_ref, emb_ref, wt_ref, b_ref, out_ref):
    del prev_ref
    out_ref[...] = (
        jnp.dot(
            emb_ref[...].astype(jnp.bfloat16),
            wt_ref[...],
            preferred_element_type=jnp.float32,
        )
        + b_ref[...]
    )


def _tc_project_chunk(k, out_prev, emb_k, Wt, b2d):
    return pl.pallas_call(
        _mm_body,
        grid=(MM_STEPS,),
        in_specs=[
            pl.BlockSpec(memory_space=pl.ANY),
            pl.BlockSpec((MM_BLK, DIM), lambda i: (i, 0)),
            pl.BlockSpec((DIM, DIM), lambda i: (0, 0)),
            pl.BlockSpec((1, DIM), lambda i: (0, 0)),
        ],
        out_specs=pl.BlockSpec((MM_BLK, DIM), lambda i, k=k: (k * MM_STEPS + i, 0)),
        out_shape=jax.ShapeDtypeStruct((N, DIM), jnp.float32),
        input_output_aliases={0: 0},
    )(out_prev, emb_k, Wt, b2d)


def _mm_first_body(emb_ref, wt_ref, b_ref, out_ref):
    out_ref[...] = (
        jnp.dot(
            emb_ref[...].astype(jnp.bfloat16),
            wt_ref[...],
            preferred_element_type=jnp.float32,
        )
        + b_ref[...]
    )


def _tc_project_first(emb_k, Wt, b2d):
    return pl.pallas_call(
        _mm_first_body,
        grid=(MM_STEPS,),
        in_specs=[
            pl.BlockSpec((MM_BLK, DIM), lambda i: (i, 0)),
            pl.BlockSpec((DIM, DIM), lambda i: (0, 0)),
            pl.BlockSpec((1, DIM), lambda i: (0, 0)),
        ],
        out_specs=pl.BlockSpec((MM_BLK, DIM), lambda i: (i, 0)),
        out_shape=jax.ShapeDtypeStruct((N, DIM), jnp.float32),
    )(emb_k, Wt, b2d)


def kernel(news_ids, news_categ, table, W, b):
    del news_categ
    ids = news_ids.T.reshape(K, NW, NCHUNK, CHUNK).astype(jnp.int32)
    Wt = W.T.astype(jnp.bfloat16)
    b2d = b.reshape(1, DIM)
    embs = [_gather_call(ids[k], table) for k in range(K)]
    out = _tc_project_first(embs[0], Wt, b2d)
    for k in range(1, K):
        out = _tc_project_chunk(k, out, embs[k], Wt, b2d)
    return out.reshape(L, B, DIM).transpose(1, 0, 2)
